# baseline (speedup 1.0000x reference)
import functools

import jax
import jax.numpy as jnp
from jax.experimental import pallas as pl
from jax.experimental.pallas import tpu as pltpu

TOPK = 16
QB = 256
KB = 2048
NG = 8
SW = KB // NG

_NEG_INF = float("-inf")
_BIG_IDX = 2**30


def _prenorm_body(k_ref, out_ref):
    k = k_ref[...]
    ss = jnp.sum(k * k, axis=1, keepdims=True)
    out_ref[...] = k / (jnp.sqrt(ss) + 1e-12)


def _knn_body(n_total, n_kb, q_ref, k_ref, vals_ref, idx_ref,
              qn_ref, rv_ref, ri_ref, vscr_ref):
    kb = pl.program_id(1)

    @pl.when(kb == 0)
    def _init():
        q = q_ref[...]
        ss = jnp.sum(q * q, axis=1, keepdims=True)
        qn_ref[...] = q / (jnp.sqrt(ss) + 1e-12)
        rv_ref[...] = jnp.full((QB, TOPK), _NEG_INF, jnp.float32)
        ri_ref[...] = jnp.zeros((QB, TOPK), jnp.float32)

    qn = qn_ref[...]
    k = k_ref[...]
    s = jax.lax.dot_general(qn, k, (((1,), (1,)), ((), ())),
                            preferred_element_type=jnp.float32)
    gcol_i = jax.lax.broadcasted_iota(jnp.int32, (QB, KB), 1) + kb * KB
    s = jnp.where(gcol_i < n_total, s, _NEG_INF)
    vscr_ref[...] = s
    m0 = jnp.concatenate(
        [jnp.max(s[:, p * SW:(p + 1) * SW], axis=1, keepdims=True)
         for p in range(NG)], axis=1)
    go0 = jnp.any(m0 >= rv_ref[:, TOPK - 1:TOPK])

    def body(carry):
        _, m8 = carry
        nm = []
        for p in range(NG):
            v = vscr_ref[:, p * SW:(p + 1) * SW]
            mp = m8[:, p:p + 1]
            gc = (jax.lax.broadcasted_iota(jnp.int32, (QB, SW), 1)
                  + (kb * KB + p * SW))
            c = jnp.min(jnp.where(v == mp, gc, _BIG_IDX), axis=1,
                        keepdims=True)
            v2 = jnp.where(gc == c, _NEG_INF, v)
            vscr_ref[:, p * SW:(p + 1) * SW] = v2
            nm.append(jnp.max(v2, axis=1, keepdims=True))
            cf = c.astype(jnp.float32)
            rv = rv_ref[...]
            ri = ri_ref[...]
            shv = jnp.concatenate(
                [jnp.full((QB, 1), jnp.inf, jnp.float32),
                 rv[:, :TOPK - 1]], axis=1)
            shi = jnp.concatenate(
                [jnp.zeros((QB, 1), jnp.float32), ri[:, :TOPK - 1]],
                axis=1)
            kp = (rv > mp) | ((rv == mp) & (ri < cf))
            kps = (shv > mp) | ((shv == mp) & (shi < cf))
            rv_ref[...] = jnp.where(kp, rv, jnp.where(kps, mp, shv))
            ri_ref[...] = jnp.where(kp, ri, jnp.where(kps, cf, shi))
        m8n = jnp.concatenate(nm, axis=1)
        go2 = jnp.any(m8n >= rv_ref[:, TOPK - 1:TOPK])
        return go2, m8n

    jax.lax.while_loop(lambda cy: cy[0], body, (go0, m0))

    @pl.when(kb == n_kb - 1)
    def _out():
        vals_ref[...] = rv_ref[...]
        idx_ref[...] = ri_ref[...].astype(jnp.int32)


@jax.jit
def kernel(queries, keys):
    q_n, d = queries.shape
    n = keys.shape[0]
    n_pad = pl.cdiv(n, KB) * KB
    n_kb = n_pad // KB
    n_qb = q_n // QB

    kpad = jnp.pad(keys, ((0, n_pad - n), (0, 0)))
    kn = pl.pallas_call(
        _prenorm_body,
        grid=(n_kb,),
        in_specs=[pl.BlockSpec((KB, d), lambda i: (i, 0))],
        out_specs=pl.BlockSpec((KB, d), lambda i: (i, 0)),
        out_shape=jax.ShapeDtypeStruct((n_pad, d), jnp.float32),
    )(kpad)

    vals, idx = pl.pallas_call(
        functools.partial(_knn_body, n, n_kb),
        grid=(n_qb, n_kb),
        in_specs=[
            pl.BlockSpec((QB, d), lambda qb, kb: (qb, 0)),
            pl.BlockSpec((KB, d), lambda qb, kb: (kb, 0)),
        ],
        out_specs=[
            pl.BlockSpec((QB, TOPK), lambda qb, kb: (qb, 0)),
            pl.BlockSpec((QB, TOPK), lambda qb, kb: (qb, 0)),
        ],
        out_shape=[
            jax.ShapeDtypeStruct((q_n, TOPK), jnp.float32),
            jax.ShapeDtypeStruct((q_n, TOPK), jnp.int32),
        ],
        scratch_shapes=[
            pltpu.VMEM((QB, d), jnp.float32),
            pltpu.VMEM((QB, TOPK), jnp.float32),
            pltpu.VMEM((QB, TOPK), jnp.float32),
            pltpu.VMEM((QB, KB), jnp.float32),
        ],
    )(queries, kn)
    return vals, idx


# speedup vs baseline: 1.8316x; 1.8316x over previous
import functools

import jax
import jax.numpy as jnp
from jax.experimental import pallas as pl
from jax.experimental.pallas import tpu as pltpu

TOPK = 16
QB = 512
KB = 2048
SW = 512

_NEG_INF = float("-inf")
_BIG_IDX = 2**30


def _prenorm_body(k_ref, out_ref):
    k = k_ref[...]
    ss = jnp.sum(k * k, axis=1, keepdims=True)
    out_ref[...] = k / (jnp.sqrt(ss) + 1e-12)


def _insert(rv, ri, mp, cf):
    shv = jnp.concatenate(
        [jnp.full((QB, 1), jnp.inf, jnp.float32), rv[:, :TOPK - 1]], axis=1)
    shi = jnp.concatenate(
        [jnp.zeros((QB, 1), jnp.float32), ri[:, :TOPK - 1]], axis=1)
    kp = (rv > mp) | ((rv == mp) & (ri < cf))
    kps = (shv > mp) | ((shv == mp) & (shi < cf))
    nrv = jnp.where(kp, rv, jnp.where(kps, mp, shv))
    nri = jnp.where(kp, ri, jnp.where(kps, cf, shi))
    return nrv, nri


def _knn_body(n_total, n_kb, q_ref, k_ref, vals_ref, idx_ref,
              qn_ref, rv_ref, ri_ref, vscr_ref):
    kb = pl.program_id(1)

    @pl.when(kb == 0)
    def _init():
        q = q_ref[...]
        ss = jnp.sum(q * q, axis=1, keepdims=True)
        qn_ref[...] = q / (jnp.sqrt(ss) + 1e-12)
        rv_ref[...] = jnp.full((QB, TOPK), _NEG_INF, jnp.float32)
        ri_ref[...] = jnp.zeros((QB, TOPK), jnp.float32)

    qn = qn_ref[...]
    k = k_ref[...]
    s = jax.lax.dot_general(qn, k, (((1,), (1,)), ((), ())),
                            preferred_element_type=jnp.float32)
    gcol_i = jax.lax.broadcasted_iota(jnp.int32, (QB, KB), 1) + kb * KB
    s = jnp.where(gcol_i < n_total, s, _NEG_INF)

    for t in range(KB // SW):
        sv = s[:, t * SW:(t + 1) * SW]
        gc = (jax.lax.broadcasted_iota(jnp.int32, (QB, SW), 1)
              + (kb * KB + t * SW))
        m0 = jnp.max(sv, axis=1, keepdims=True)
        go0 = jnp.any(m0 >= rv_ref[:, TOPK - 1:TOPK])

        @pl.when(go0)
        def _stage():
            vscr_ref[...] = sv

        def body(carry):
            _, m = carry
            v = vscr_ref[...]
            c = jnp.min(jnp.where(v == m, gc, _BIG_IDX), axis=1,
                        keepdims=True)
            v2 = jnp.where(gc == c, _NEG_INF, v)
            vscr_ref[...] = v2
            nrv, nri = _insert(rv_ref[...], ri_ref[...], m,
                               c.astype(jnp.float32))
            rv_ref[...] = nrv
            ri_ref[...] = nri
            m2 = jnp.max(v2, axis=1, keepdims=True)
            go2 = jnp.any(m2 >= nrv[:, TOPK - 1:TOPK])
            return go2, m2

        jax.lax.while_loop(lambda cy: cy[0], body, (go0, m0))

    @pl.when(kb == n_kb - 1)
    def _out():
        vals_ref[...] = rv_ref[...]
        idx_ref[...] = ri_ref[...].astype(jnp.int32)


@jax.jit
def kernel(queries, keys):
    q_n, d = queries.shape
    n = keys.shape[0]
    n_pad = pl.cdiv(n, KB) * KB
    n_kb = n_pad // KB
    n_qb = q_n // QB

    kpad = jnp.pad(keys, ((0, n_pad - n), (0, 0)))
    kn = pl.pallas_call(
        _prenorm_body,
        grid=(n_kb,),
        in_specs=[pl.BlockSpec((KB, d), lambda i: (i, 0))],
        out_specs=pl.BlockSpec((KB, d), lambda i: (i, 0)),
        out_shape=jax.ShapeDtypeStruct((n_pad, d), jnp.float32),
    )(kpad)

    vals, idx = pl.pallas_call(
        functools.partial(_knn_body, n, n_kb),
        grid=(n_qb, n_kb),
        in_specs=[
            pl.BlockSpec((QB, d), lambda qb, kb: (qb, 0)),
            pl.BlockSpec((KB, d), lambda qb, kb: (kb, 0)),
        ],
        out_specs=[
            pl.BlockSpec((QB, TOPK), lambda qb, kb: (qb, 0)),
            pl.BlockSpec((QB, TOPK), lambda qb, kb: (qb, 0)),
        ],
        out_shape=[
            jax.ShapeDtypeStruct((q_n, TOPK), jnp.float32),
            jax.ShapeDtypeStruct((q_n, TOPK), jnp.int32),
        ],
        scratch_shapes=[
            pltpu.VMEM((QB, d), jnp.float32),
            pltpu.VMEM((QB, TOPK), jnp.float32),
            pltpu.VMEM((QB, TOPK), jnp.float32),
            pltpu.VMEM((QB, SW), jnp.float32),
        ],
    )(queries, kn)
    return vals, idx


# speedup vs baseline: 4.0308x; 2.2007x over previous
import functools

import jax
import jax.numpy as jnp
from jax import lax
from jax.experimental import pallas as pl
from jax.experimental.pallas import tpu as pltpu
from jax.experimental.pallas import tpu_sc as plsc

TOPK = 16
QB = 512
KB = 2048
CH = 128
NCH_STEP = KB // CH

_NEG_INF = float("-inf")
_BIG = 3.0e7


def _phase_a_body(n_total, q_ref, k_ref, s_ref, cm_ref):
    kb = pl.program_id(1)
    s = jax.lax.dot_general(q_ref[...], k_ref[...], (((1,), (1,)), ((), ())),
                            preferred_element_type=jnp.float32)
    gcol_i = jax.lax.broadcasted_iota(jnp.int32, (QB, KB), 1) + kb * KB
    s = jnp.where(gcol_i < n_total, s, _NEG_INF)
    s_ref[...] = s
    cm_ref[0, :, :] = jnp.concatenate(
        [jnp.max(s[:, i * CH:(i + 1) * CH], axis=1, keepdims=True)
         for i in range(NCH_STEP)], axis=1)


def _phase_b_body(nch, n_kb, cm_ref, flat_ref):
    qb = pl.program_id(0)
    v = jnp.concatenate([cm_ref[i, :, :] for i in range(n_kb)], axis=1)
    cidx = jax.lax.broadcasted_iota(jnp.int32, (QB, nch), 1).astype(jnp.float32)
    rowq = (jax.lax.broadcasted_iota(jnp.int32, (QB, TOPK), 0)
            + qb * QB)
    cs = []
    for _ in range(TOPK):
        m = jnp.max(v, axis=1, keepdims=True)
        c = jnp.min(jnp.where(v == m, cidx, _BIG), axis=1, keepdims=True)
        cs.append(c)
        v = jnp.where(cidx == c, _NEG_INF, v)
    cid = jnp.concatenate(cs, axis=1).astype(jnp.int32)
    flat_ref[...] = rowq * nch + cid


def _phase_d_body(nch, g_ref, flat_ref, vals_ref, idx_ref):
    qb = pl.program_id(0)
    rowq = (jax.lax.broadcasted_iota(jnp.int32, (QB, TOPK), 0)
            + qb * QB)
    cid = (flat_ref[...] - rowq * nch).astype(jnp.float32)
    erow = jax.lax.broadcasted_iota(jnp.int32, (TOPK, TOPK * CH), 0)
    ecol = jax.lax.broadcasted_iota(jnp.int32, (TOPK, TOPK * CH), 1) // CH
    emat = (erow == ecol).astype(jnp.float32)
    cexp = jax.lax.dot_general(cid, emat, (((1,), (0,)), ((), ())),
                               precision=jax.lax.Precision.HIGHEST,
                               preferred_element_type=jnp.float32)
    lane = (jax.lax.broadcasted_iota(jnp.int32, (QB, TOPK * CH), 1)
            % CH).astype(jnp.float32)
    gidx = cexp * CH + lane
    v = g_ref[...]
    ms, cs = [], []
    for _ in range(TOPK):
        m = jnp.max(v, axis=1, keepdims=True)
        c = jnp.min(jnp.where(v == m, gidx, _BIG), axis=1, keepdims=True)
        ms.append(m)
        cs.append(c)
        v = jnp.where(gidx == c, _NEG_INF, v)
    vals_ref[...] = jnp.concatenate(ms, axis=1)
    idx_ref[...] = jnp.concatenate(cs, axis=1).astype(jnp.int32)


def _gather_chunks(scores_flat, flat_ids):
    b_total = flat_ids.shape[0]
    d = scores_flat.shape[1]
    info = plsc.get_sparse_core_info()
    nw = info.num_cores * info.num_subcores
    bpw = b_total // nw
    rows = 256
    nloop = bpw // rows

    def body(table_hbm, idx_hbm, out_hbm, idxc_v, rows_v, sem):
        wid = lax.axis_index("s") * info.num_cores + lax.axis_index("c")
        base = wid * bpw

        @pl.loop(0, nloop)
        def _g(i):
            off = base + i * rows
            pltpu.sync_copy(idx_hbm.at[pl.ds(off, rows)], idxc_v)
            pltpu.async_copy(table_hbm.at[idxc_v], rows_v, sem).wait()
            pltpu.sync_copy(rows_v, out_hbm.at[pl.ds(off, rows)])

    return pl.kernel(
        body,
        mesh=plsc.VectorSubcoreMesh(core_axis_name="c", subcore_axis_name="s"),
        out_type=jax.ShapeDtypeStruct((b_total, d), jnp.float32),
        scratch_types=[
            pltpu.VMEM((rows,), jnp.int32),
            pltpu.VMEM((rows, d), jnp.float32),
            pltpu.SemaphoreType.DMA,
        ],
    )(scores_flat, flat_ids)


@jax.jit
def kernel(queries, keys):
    q_n, d = queries.shape
    n = keys.shape[0]
    n_pad = pl.cdiv(n, KB) * KB
    n_kb = n_pad // KB
    n_qb = q_n // QB
    nch = n_pad // CH

    qn = queries / (jnp.linalg.norm(queries, axis=-1, keepdims=True) + 1e-12)
    knorm = keys / (jnp.linalg.norm(keys, axis=-1, keepdims=True) + 1e-12)
    kn = jnp.pad(knorm, ((0, n_pad - n), (0, 0)))

    scores, cm = pl.pallas_call(
        functools.partial(_phase_a_body, n),
        grid=(n_qb, n_kb),
        in_specs=[
            pl.BlockSpec((QB, d), lambda qb, kb: (qb, 0)),
            pl.BlockSpec((KB, d), lambda qb, kb: (kb, 0)),
        ],
        out_specs=[
            pl.BlockSpec((QB, KB), lambda qb, kb: (qb, kb)),
            pl.BlockSpec((1, QB, NCH_STEP), lambda qb, kb: (kb, qb, 0)),
        ],
        out_shape=[
            jax.ShapeDtypeStruct((q_n, n_pad), jnp.float32),
            jax.ShapeDtypeStruct((n_kb, q_n, NCH_STEP), jnp.float32),
        ],
    )(qn, kn)

    flat = pl.pallas_call(
        functools.partial(_phase_b_body, nch, n_kb),
        grid=(n_qb,),
        in_specs=[pl.BlockSpec((n_kb, QB, NCH_STEP), lambda qb: (0, qb, 0))],
        out_specs=pl.BlockSpec((QB, TOPK), lambda qb: (qb, 0)),
        out_shape=jax.ShapeDtypeStruct((q_n, TOPK), jnp.int32),
    )(cm)

    scores_flat = scores.reshape(q_n * nch, CH)
    g = _gather_chunks(scores_flat, flat.reshape(-1)).reshape(q_n, TOPK * CH)

    vals, idx = pl.pallas_call(
        functools.partial(_phase_d_body, nch),
        grid=(n_qb,),
        in_specs=[
            pl.BlockSpec((QB, TOPK * CH), lambda qb: (qb, 0)),
            pl.BlockSpec((QB, TOPK), lambda qb: (qb, 0)),
        ],
        out_specs=[
            pl.BlockSpec((QB, TOPK), lambda qb: (qb, 0)),
            pl.BlockSpec((QB, TOPK), lambda qb: (qb, 0)),
        ],
        out_shape=[
            jax.ShapeDtypeStruct((q_n, TOPK), jnp.float32),
            jax.ShapeDtypeStruct((q_n, TOPK), jnp.int32),
        ],
    )(g, flat)
    return vals, idx
